# baseline (speedup 1.0000x reference)
import functools

import jax
import jax.numpy as jnp
from jax import lax
from jax.experimental import pallas as pl
from jax.experimental.pallas import tpu as pltpu
from jax.experimental.pallas import tpu_sc as plsc

_DIM = 64
_CHUNK = 128
_NC = 2
_NS = 16
_NW = _NC * _NS


@functools.lru_cache(maxsize=None)
def _build_gather(B: int):
    assert B % (_NW * _CHUNK) == 0
    b_per_w = B // _NW
    n_chunks = b_per_w // _CHUNK
    mesh = plsc.VectorSubcoreMesh(core_axis_name="c", subcore_axis_name="s")

    @functools.partial(
        pl.kernel,
        mesh=mesh,
        out_type=jax.ShapeDtypeStruct((B, _DIM), jnp.float32),
        scratch_types=[
            pltpu.VMEM((n_chunks, _CHUNK), jnp.int32),
            pltpu.VMEM((2, _CHUNK, _DIM), jnp.float32),
            pltpu.SemaphoreType.DMA,
            pltpu.SemaphoreType.DMA,
        ],
        compiler_params=pltpu.CompilerParams(use_tc_tiling_on_sc=False),
    )
    def gather_kernel(idx_hbm, table_hbm, out_hbm, idx_v, rows_v, sem0, sem1):
        wid = lax.axis_index("s") * _NC + lax.axis_index("c")
        base = wid * b_per_w
        pltpu.sync_copy(idx_hbm.at[wid], idx_v)
        sems = (sem0, sem1)

        for b in range(2):
            pltpu.async_copy(table_hbm.at[idx_v.at[b]], rows_v.at[b], sems[b])

        def body(i, carry):
            for b in range(2):
                ch = 2 * i + b
                pltpu.make_async_copy(
                    table_hbm.at[idx_v.at[ch]], rows_v.at[b], sems[b]
                ).wait()
                pltpu.sync_copy(
                    rows_v.at[b], out_hbm.at[pl.ds(base + ch * _CHUNK, _CHUNK)]
                )
                nxt = ch + 2

                @pl.when(nxt < n_chunks)
                def _():
                    pltpu.async_copy(
                        table_hbm.at[idx_v.at[nxt]], rows_v.at[b], sems[b]
                    )
            return carry

        lax.fori_loop(0, n_chunks // 2, body, 0)

    return gather_kernel


def kernel(inputs, embeddings):
    rows, cols = inputs.shape
    B = rows * cols
    idx = inputs.reshape(_NW, B // (_NW * _CHUNK), _CHUNK).astype(jnp.int32)
    out = _build_gather(B)(idx, embeddings)
    return out.reshape(rows, cols, _DIM)


# speedup vs baseline: 1.0046x; 1.0046x over previous
import functools

import jax
import jax.numpy as jnp
from jax import lax
from jax.experimental import pallas as pl
from jax.experimental.pallas import tpu as pltpu
from jax.experimental.pallas import tpu_sc as plsc

_DIM = 64
_CHUNK = 128
_NC = 2
_NS = 16
_NW = _NC * _NS


_NBUF = 8


@functools.lru_cache(maxsize=None)
def _build_gather(B: int):
    assert B % (_NW * _CHUNK) == 0
    b_per_w = B // _NW
    n_chunks = b_per_w // _CHUNK
    mesh = plsc.VectorSubcoreMesh(core_axis_name="c", subcore_axis_name="s")

    @functools.partial(
        pl.kernel,
        mesh=mesh,
        out_type=jax.ShapeDtypeStruct((B, _DIM), jnp.float32),
        scratch_types=[
            pltpu.VMEM((n_chunks, _CHUNK), jnp.int32),
            pltpu.VMEM((_NBUF, _CHUNK, _DIM), jnp.float32),
        ]
        + [pltpu.SemaphoreType.DMA] * (2 * _NBUF),
        compiler_params=pltpu.CompilerParams(use_tc_tiling_on_sc=False),
    )
    def gather_kernel(idx_hbm, table_hbm, out_hbm, idx_v, rows_v, *sems):
        gsem = sems[:_NBUF]
        wsem = sems[_NBUF:]
        wid = lax.axis_index("s") * _NC + lax.axis_index("c")
        base = wid * b_per_w
        pltpu.sync_copy(idx_hbm.at[wid], idx_v)

        def gather(ch, b):
            return pltpu.async_copy(
                table_hbm.at[idx_v.at[ch]], rows_v.at[b], gsem[b]
            )

        def write(ch, b):
            return pltpu.async_copy(
                rows_v.at[b], out_hbm.at[pl.ds(base + ch * _CHUNK, _CHUNK)],
                wsem[b],
            )

        for ch in range(_NBUF):
            gather(ch, ch)
        for ch in range(n_chunks):
            b = ch % _NBUF
            pltpu.make_async_copy(
                table_hbm.at[idx_v.at[ch]], rows_v.at[b], gsem[b]
            ).wait()
            w = write(ch, b)
            nxt = ch + _NBUF
            if nxt < n_chunks:
                w.wait()
                gather(nxt, b)
        for ch in range(n_chunks - _NBUF, n_chunks):
            b = ch % _NBUF
            pltpu.make_async_copy(
                rows_v.at[b], out_hbm.at[pl.ds(base + ch * _CHUNK, _CHUNK)],
                wsem[b],
            ).wait()

    return gather_kernel


def kernel(inputs, embeddings):
    rows, cols = inputs.shape
    B = rows * cols
    idx = inputs.reshape(_NW, B // (_NW * _CHUNK), _CHUNK).astype(jnp.int32)
    out = _build_gather(B)(idx, embeddings)
    return out.reshape(rows, cols, _DIM)
